# baseline (speedup 1.0000x reference)
import functools
import jax
import jax.numpy as jnp
from jax import lax
from jax.experimental import pallas as pl
from jax.experimental.pallas import tpu as pltpu
from jax.experimental.pallas import tpu_sc as plsc

N0, E, DIM, NCLS = 10000, 320000, 128, 10
K1, K2 = 5000, 2500
N0P, N1P, N2P = 10240, 5120, 2560
CHUNK = 128
NC, NS = 2, 16
NWORK = NC * NS
EP = 323584
MININT = -2147483648
MAXINT = 2147483647

_mesh = lambda: plsc.VectorSubcoreMesh(core_axis_name="c", subcore_axis_name="s",
                                       num_cores=NC, num_subcores=NS)


def _sc_params():
    import dataclasses
    cp = pltpu.CompilerParams()
    if "needs_layout_passes" in pltpu.CompilerParams.__dataclass_fields__:
        cp = dataclasses.replace(cp, needs_layout_passes=False)
    return cp


def _zero16f():
    return jnp.zeros((16,), jnp.float32)


@functools.lru_cache(None)
def _edge_agg(np_, with_deg):
    nchunks = EP // (NWORK * CHUNK)
    rows_pt = np_ // NS
    Z = 32
    outs = [jax.ShapeDtypeStruct((2, np_, DIM), jnp.float32)]
    if with_deg:
        outs.append(jax.ShapeDtypeStruct((NWORK, np_), jnp.float32))
    scratch = [
        pltpu.VMEM((CHUNK,), jnp.int32),
        pltpu.VMEM((CHUNK,), jnp.int32),
        pltpu.VMEM((CHUNK, DIM), jnp.float32),
        pltpu.VMEM((np_,), jnp.float32),
        pltpu.VMEM_SHARED((np_, DIM), jnp.float32),
        pltpu.SemaphoreType.DMA,
    ]

    def body(xp, srcp, dstp, *rest):
        if with_deg:
            agg_out, deg_out, sidx, didx, rows, degl, acc, sem = rest
        else:
            agg_out, sidx, didx, rows, degl, acc, sem = rest
        c = lax.axis_index("c")
        s = lax.axis_index("s")
        w = c * NS + s

        @pl.loop(0, CHUNK)
        def _(r):
            @pl.loop(0, DIM, step=16)
            def _(j):
                rows[r, pl.ds(j, 16)] = _zero16f()

        @pl.loop(0, rows_pt, step=Z)
        def _(t):
            pltpu.sync_copy(rows.at[pl.ds(0, Z)], acc.at[pl.ds(s * rows_pt + t, Z)])

        if with_deg:
            @pl.loop(0, np_, step=16)
            def _(i):
                degl[pl.ds(i, 16)] = _zero16f()

        plsc.subcore_barrier()

        epw = EP // NWORK
        base = w * epw

        @pl.loop(0, nchunks)
        def _(ch):
            off = base + ch * CHUNK
            pltpu.sync_copy(srcp.at[pl.ds(off, CHUNK)], sidx)
            pltpu.sync_copy(dstp.at[pl.ds(off, CHUNK)], didx)
            pltpu.async_copy(xp.at[sidx], rows, sem).wait()
            pltpu.sync_copy(rows, acc.at[didx], add=True)
            if with_deg:
                @pl.loop(0, CHUNK, step=16)
                def _(j):
                    dvec = didx[pl.ds(j, 16)]
                    plsc.addupdate_scatter(degl, [dvec], jnp.ones((16,), jnp.float32))

        plsc.subcore_barrier()

        @pl.loop(0, rows_pt, step=Z)
        def _(t):
            r0 = s * rows_pt + t
            pltpu.sync_copy(acc.at[pl.ds(r0, Z)], agg_out.at[c].at[pl.ds(r0, Z)])

        if with_deg:
            pltpu.sync_copy(degl, deg_out.at[w])

    return pl.kernel(body, out_type=tuple(outs) if with_deg else outs[0],
                     mesh=_mesh(), scratch_types=scratch,
                     compiler_params=_sc_params())


@functools.lru_cache(None)
def _score_scatter(np_):
    nchunks = EP // (NWORK * CHUNK)
    scratch = [
        pltpu.VMEM((CHUNK,), jnp.int32),
        pltpu.VMEM((CHUNK,), jnp.int32),
        pltpu.VMEM((np_,), jnp.float32),
        pltpu.VMEM((np_,), jnp.float32),
        pltpu.VMEM((np_,), jnp.float32),
    ]

    def body(srcp, dstp, dinv_h, h_h, sc_out, sidx, didx, dinvl, hl, accl):
        c = lax.axis_index("c")
        s = lax.axis_index("s")
        w = c * NS + s
        pltpu.sync_copy(dinv_h, dinvl)
        pltpu.sync_copy(h_h, hl)

        @pl.loop(0, np_, step=16)
        def _(i):
            accl[pl.ds(i, 16)] = _zero16f()

        base = w * (EP // NWORK)

        @pl.loop(0, nchunks)
        def _(ch):
            off = base + ch * CHUNK
            pltpu.sync_copy(srcp.at[pl.ds(off, CHUNK)], sidx)
            pltpu.sync_copy(dstp.at[pl.ds(off, CHUNK)], didx)

            @pl.loop(0, CHUNK, step=16)
            def _(j):
                sv = sidx[pl.ds(j, 16)]
                dv = didx[pl.ds(j, 16)]
                hs = plsc.load_gather(hl, [sv])
                da = plsc.load_gather(dinvl, [sv])
                db = plsc.load_gather(dinvl, [dv])
                plsc.addupdate_scatter(accl, [dv], hs * da * db)

        pltpu.sync_copy(accl, sc_out.at[w])

    return pl.kernel(body, out_type=jax.ShapeDtypeStruct((NWORK, np_), jnp.float32),
                     mesh=_mesh(), scratch_types=scratch,
                     compiler_params=_sc_params())


@functools.lru_cache(None)
def _edge_remap(np_old, trash):
    nchunks = EP // (NWORK * CHUNK)
    scratch = [
        pltpu.VMEM((np_old,), jnp.int32),
        pltpu.VMEM((CHUNK,), jnp.int32),
        pltpu.VMEM((CHUNK,), jnp.int32),
        pltpu.VMEM((CHUNK,), jnp.int32),
        pltpu.VMEM((CHUNK,), jnp.int32),
    ]

    def body(srcp, dstp, nm_h, ns_out, nd_out, nml, sidx, didx, so, do):
        c = lax.axis_index("c")
        s = lax.axis_index("s")
        w = c * NS + s
        pltpu.sync_copy(nm_h, nml)
        base = w * (EP // NWORK)

        @pl.loop(0, nchunks)
        def _(ch):
            off = base + ch * CHUNK
            pltpu.sync_copy(srcp.at[pl.ds(off, CHUNK)], sidx)
            pltpu.sync_copy(dstp.at[pl.ds(off, CHUNK)], didx)

            @pl.loop(0, CHUNK, step=16)
            def _(j):
                sv = sidx[pl.ds(j, 16)]
                dv = didx[pl.ds(j, 16)]
                ns = plsc.load_gather(nml, [sv])
                nd = plsc.load_gather(nml, [dv])
                valid = (ns >= 0) & (nd >= 0)
                so[pl.ds(j, 16)] = jnp.where(valid, ns, trash)
                do[pl.ds(j, 16)] = jnp.where(valid, nd, trash)

            pltpu.sync_copy(so, ns_out.at[pl.ds(off, CHUNK)])
            pltpu.sync_copy(do, nd_out.at[pl.ds(off, CHUNK)])

    out = (jax.ShapeDtypeStruct((EP,), jnp.int32), jax.ShapeDtypeStruct((EP,), jnp.int32))
    return pl.kernel(body, out_type=out, mesh=_mesh(), scratch_types=scratch,
                     compiler_params=_sc_params())


@functools.lru_cache(None)
def _row_gather(np_out):
    nch = np_out // CHUNK
    outer = (nch + NWORK - 1) // NWORK
    scratch = [
        pltpu.VMEM((CHUNK,), jnp.int32),
        pltpu.VMEM((CHUNK, DIM), jnp.float32),
        pltpu.SemaphoreType.DMA,
    ]

    def body(xp, idx_h, out, idxv, rows, sem):
        c = lax.axis_index("c")
        s = lax.axis_index("s")
        w = c * NS + s

        @pl.loop(0, outer)
        def _(t):
            ch = t * NWORK + w

            @pl.when(ch < nch)
            def _():
                off = ch * CHUNK
                pltpu.sync_copy(idx_h.at[pl.ds(off, CHUNK)], idxv)
                pltpu.async_copy(xp.at[idxv], rows, sem).wait()
                pltpu.sync_copy(rows, out.at[pl.ds(off, CHUNK)])

    return pl.kernel(body, out_type=jax.ShapeDtypeStruct((np_out, DIM), jnp.float32),
                     mesh=_mesh(), scratch_types=scratch,
                     compiler_params=_sc_params())


@functools.lru_cache(None)
def _select_compact(np_, kp, k, trash_next, n_src):
    scratch = [
        pltpu.VMEM((np_,), jnp.float32),
        pltpu.VMEM((np_,), jnp.int32),
        pltpu.VMEM((np_,), jnp.int32),
        pltpu.VMEM((kp,), jnp.int32),
        pltpu.VMEM((kp,), jnp.float32),
        pltpu.VMEM((32,), jnp.int32),
    ]

    def body(score_h, selp_h, nm_out, cnm_out, perm_out, vals_out,
             scl, nml, cnml, perml, valsl, selps):
        c = lax.axis_index("c")
        s = lax.axis_index("s")
        w = c * NS + s

        @pl.when(w == 0)
        def _():
            pltpu.sync_copy(score_h, scl)
            pltpu.sync_copy(selp_h, selps)
            tkey = selps[pl.ds(0, 16)]
            mi = selps[pl.ds(16, 16)]

            @pl.loop(0, kp, step=16)
            def _(i):
                perml[pl.ds(i, 16)] = jnp.full((16,), n_src, jnp.int32)
                valsl[pl.ds(i, 16)] = _zero16f()

            def step(i, p):
                sv = scl[pl.ds(i * 16, 16)]
                b = lax.bitcast_convert_type(sv, jnp.int32)
                mneg = lax.shift_right_arithmetic(b, 31)
                key = b ^ (mneg & MAXINT)
                gid = jnp.arange(16, dtype=jnp.int32) + i * 16
                selm = (key > tkey) | ((key == tkey) & (gid <= mi))
                cnt = jnp.cumsum(selm.astype(jnp.int32))
                ranks = p + cnt - 1
                nml[pl.ds(i * 16, 16)] = jnp.where(selm, ranks, -1)
                cnml[pl.ds(i * 16, 16)] = jnp.where(selm, ranks, trash_next)
                sidx = jnp.minimum(ranks, kp - 1)
                plsc.store_scatter(perml, [sidx], gid, mask=selm)
                plsc.store_scatter(valsl, [sidx], sv, mask=selm)
                return p + jnp.sum(selm.astype(jnp.int32))

            lax.fori_loop(0, np_ // 16, step, jnp.int32(0))
            pltpu.sync_copy(nml, nm_out)
            pltpu.sync_copy(cnml, cnm_out)
            pltpu.sync_copy(perml, perm_out)
            pltpu.sync_copy(valsl, vals_out)

    out = (jax.ShapeDtypeStruct((np_,), jnp.int32),
           jax.ShapeDtypeStruct((np_,), jnp.int32),
           jax.ShapeDtypeStruct((kp,), jnp.int32),
           jax.ShapeDtypeStruct((kp,), jnp.float32))
    return pl.kernel(body, out_type=out, mesh=_mesh(), scratch_types=scratch,
                     compiler_params=_sc_params())


def _tc_gin(base, aggp, W1, b1, W2, b2, g, bb, n_real, extras=None):
    np_ = base.shape[0]
    BR = 512
    grid = (np_ // BR,)

    def body(base_r, agg_r, w1_r, b1_r, w2_r, b2_r, g_r, bb_r, *rest):
        if extras is not None:
            deg_r, pw_r, out_r, h_r, dinv_r, deg_o = rest
        else:
            (out_r,) = rest
        pid = pl.program_id(0)
        h = base_r[...] + agg_r[0] + agg_r[1]
        y = jnp.maximum(jnp.dot(h, w1_r[...], preferred_element_type=jnp.float32) + b1_r[...], 0.0)
        y = jnp.maximum(jnp.dot(y, w2_r[...], preferred_element_type=jnp.float32) + b2_r[...], 0.0)
        y = y * g_r[...] + bb_r[...]
        rid = lax.broadcasted_iota(jnp.int32, (BR, DIM), 0) + pid * BR
        y = jnp.where(rid < n_real, y, 0.0)
        out_r[...] = y
        if extras is not None:
            deg = jnp.sum(deg_r[...], axis=0)
            deg_o[...] = deg
            dinv_r[...] = lax.rsqrt(deg + 1.0)
            h_r[...] = jnp.dot(y, pw_r[...], preferred_element_type=jnp.float32)

    in_specs = [
        pl.BlockSpec((BR, DIM), lambda i: (i, 0)),
        pl.BlockSpec((2, BR, DIM), lambda i: (0, i, 0)),
        pl.BlockSpec((DIM, DIM), lambda i: (0, 0)),
        pl.BlockSpec((1, DIM), lambda i: (0, 0)),
        pl.BlockSpec((DIM, DIM), lambda i: (0, 0)),
        pl.BlockSpec((1, DIM), lambda i: (0, 0)),
        pl.BlockSpec((1, DIM), lambda i: (0, 0)),
        pl.BlockSpec((1, DIM), lambda i: (0, 0)),
    ]
    out_shapes = [jax.ShapeDtypeStruct((np_, DIM), jnp.float32)]
    out_specs = [pl.BlockSpec((BR, DIM), lambda i: (i, 0))]
    args = [base, aggp, W1, b1.reshape(1, DIM), W2, b2.reshape(1, DIM),
            g.reshape(1, DIM), bb.reshape(1, DIM)]
    if extras is not None:
        degp, pw = extras
        in_specs += [pl.BlockSpec((NWORK, BR), lambda i: (0, i)),
                     pl.BlockSpec((DIM, 1), lambda i: (0, 0))]
        args += [degp, pw.reshape(DIM, 1)]
        out_shapes += [jax.ShapeDtypeStruct((np_, 1), jnp.float32),
                       jax.ShapeDtypeStruct((np_,), jnp.float32),
                       jax.ShapeDtypeStruct((np_,), jnp.float32)]
        out_specs += [pl.BlockSpec((BR, 1), lambda i: (i, 0)),
                      pl.BlockSpec((BR,), lambda i: (i,)),
                      pl.BlockSpec((BR,), lambda i: (i,))]
    return pl.pallas_call(body, grid=grid, in_specs=in_specs,
                          out_specs=out_specs, out_shape=out_shapes)(*args)


def _tc_topk(scparts, dinv, h, bpad, k, n_real):
    np_ = dinv.shape[0]

    def body(sc_r, dinv_r, h_r, b_r, score_o, selp_o):
        sc = jnp.sum(sc_r[...], axis=0) + dinv_r[...] * dinv_r[...] * h_r[...] + b_r[0]
        idx = lax.broadcasted_iota(jnp.int32, (np_,), 0)
        sc = jnp.where(idx < n_real, sc, -jnp.inf)
        score_o[...] = sc
        b = lax.bitcast_convert_type(sc, jnp.int32)
        mneg = lax.shift_right_arithmetic(b, 31)
        keys = b ^ (mneg & MAXINT)
        cpos = jnp.sum((keys >= 0).astype(jnp.int32))
        lo = jnp.where(cpos >= k, jnp.int32(0), jnp.int32(MININT))
        hi = jnp.where(cpos >= k, jnp.int32(MAXINT), jnp.int32(-1))

        def step(_, lh):
            lo, hi = lh
            mid = lo + (hi - lo) // 2
            cgt = jnp.sum((keys > mid).astype(jnp.int32))
            take = cgt < k
            return jnp.where(take, lo, mid + 1), jnp.where(take, mid, hi)

        lo, hi = lax.fori_loop(0, 31, step, (lo, hi))
        t = lo
        n_gt = jnp.sum((keys > t).astype(jnp.int32))
        r = k - n_gt
        lo2, hi2 = jnp.int32(0), jnp.int32(np_ - 1)

        def step2(_, lh):
            lo2, hi2 = lh
            mid = lo2 + (hi2 - lo2) // 2
            cc = jnp.sum(((keys == t) & (idx <= mid)).astype(jnp.int32))
            take = cc >= r
            return jnp.where(take, lo2, mid + 1), jnp.where(take, mid, hi2)

        lo2, hi2 = lax.fori_loop(0, 15, step2, (lo2, hi2))
        m = lo2
        lane = lax.broadcasted_iota(jnp.int32, (32,), 0)
        selp_o[...] = jnp.where(lane < 16, t, m)

    return pl.pallas_call(
        body, grid=(1,),
        in_specs=[pl.BlockSpec((NWORK, np_), lambda i: (0, 0)),
                  pl.BlockSpec((np_,), lambda i: (0,)),
                  pl.BlockSpec((np_,), lambda i: (0,)),
                  pl.BlockSpec(memory_space=pltpu.SMEM)],
        out_specs=[pl.BlockSpec((np_,), lambda i: (0,)),
                   pl.BlockSpec((32,), lambda i: (0,))],
        out_shape=[jax.ShapeDtypeStruct((np_,), jnp.float32),
                   jax.ShapeDtypeStruct((32,), jnp.int32)],
    )(scparts, dinv, h, bpad)


def _tc_pool(xs, vals, g, bb, k):
    np_ = xs.shape[0]
    BR = 512 if np_ % 512 == 0 else 256

    def body(xs_r, v_r, g_r, bb_r, out_r):
        pid = pl.program_id(0)
        y = xs_r[...] * jnp.tanh(v_r[...]) * g_r[...] + bb_r[...]
        rid = lax.broadcasted_iota(jnp.int32, (BR, DIM), 0) + pid * BR
        out_r[...] = jnp.where(rid < k, y, 0.0)

    return pl.pallas_call(
        body, grid=(np_ // BR,),
        in_specs=[pl.BlockSpec((BR, DIM), lambda i: (i, 0)),
                  pl.BlockSpec((BR, 1), lambda i: (i, 0)),
                  pl.BlockSpec((1, DIM), lambda i: (0, 0)),
                  pl.BlockSpec((1, DIM), lambda i: (0, 0))],
        out_specs=pl.BlockSpec((BR, DIM), lambda i: (i, 0)),
        out_shape=jax.ShapeDtypeStruct((np_, DIM), jnp.float32),
    )(xs, vals.reshape(np_, 1), g.reshape(1, DIM), bb.reshape(1, DIM))


def _tc_unpool(up, sparts, nm, deg, xskip, rw, n_real):
    np_ = up.shape[0]
    BR = 512 if np_ % 512 == 0 else 256

    def body(up_r, s_r, nm_r, deg_r, xs_r, rw_r, out_r):
        pid = pl.program_id(0)
        ssum = s_r[0] + s_r[1]
        mean = ssum / jnp.maximum(deg_r[...], 1.0)
        sel = nm_r[...] >= 0
        y = jnp.where(sel, up_r[...], mean) + rw_r[0] * xs_r[...]
        rid = lax.broadcasted_iota(jnp.int32, (BR, DIM), 0) + pid * BR
        out_r[...] = jnp.where(rid < n_real, y, 0.0)

    return pl.pallas_call(
        body, grid=(np_ // BR,),
        in_specs=[pl.BlockSpec((BR, DIM), lambda i: (i, 0)),
                  pl.BlockSpec((2, BR, DIM), lambda i: (0, i, 0)),
                  pl.BlockSpec((BR, 1), lambda i: (i, 0)),
                  pl.BlockSpec((BR, 1), lambda i: (i, 0)),
                  pl.BlockSpec((BR, DIM), lambda i: (i, 0)),
                  pl.BlockSpec(memory_space=pltpu.SMEM)],
        out_specs=pl.BlockSpec((BR, DIM), lambda i: (i, 0)),
        out_shape=jax.ShapeDtypeStruct((np_, DIM), jnp.float32),
    )(up, sparts, nm.reshape(np_, 1), deg.reshape(np_, 1), xskip, rw)


def _tc_read(x, n_real):
    np_ = x.shape[0]

    def body(x_r, out_r):
        rid = lax.broadcasted_iota(jnp.int32, (np_, DIM), 0)
        xr = x_r[...]
        mx = jnp.max(jnp.where(rid < n_real, xr, -jnp.inf), axis=0)
        mn = jnp.sum(jnp.where(rid < n_real, xr, 0.0), axis=0) / n_real
        out_r[...] = jnp.concatenate([mx[None, :], mn[None, :]], axis=1)

    return pl.pallas_call(
        body, grid=(1,),
        in_specs=[pl.BlockSpec((np_, DIM), lambda i: (0, 0))],
        out_specs=pl.BlockSpec((1, 2 * DIM), lambda i: (0, 0)),
        out_shape=jax.ShapeDtypeStruct((1, 2 * DIM), jnp.float32),
    )(x)


def _tc_head(r2, r3, r4, p):
    TD = 2 * DIM

    def body(r2_r, r3_r, r4_r, a1w, a1b, a2w, a2b, a3w, a3b, g61, b61, g62, b62,
             g63, b63, aw, ab, g6, b6, lw, lb, out_r):
        def attn2(rr, w_r, b_r, g_r, bb_r):
            logit = jnp.dot(rr, w_r[...], preferred_element_type=jnp.float32) + b_r[...]
            e = jnp.exp(logit - jnp.max(logit, axis=1, keepdims=True))
            a = e / jnp.sum(e, axis=1, keepdims=True)
            scaled = jnp.concatenate([rr[:, :DIM] * a[:, 0:1], rr[:, DIM:] * a[:, 1:2]], axis=1)
            return scaled * g_r[...] + bb_r[...]

        rr2 = attn2(r2_r[...], a1w, a1b, g61, b61)
        rr3 = attn2(r3_r[...], a2w, a2b, g62, b62)
        rr4 = attn2(r4_r[...], a3w, a3b, g63, b63)
        xc = jnp.concatenate([rr2, rr3, rr4], axis=1)
        logit = jnp.dot(xc, aw[...], preferred_element_type=jnp.float32) + ab[...]
        e = jnp.exp(logit - jnp.max(logit, axis=1, keepdims=True))
        a = e / jnp.sum(e, axis=1, keepdims=True)
        xc = jnp.concatenate([xc[:, :TD] * a[:, 0:1], xc[:, TD:2 * TD] * a[:, 1:2],
                              xc[:, 2 * TD:] * a[:, 2:3]], axis=1)
        xc = xc * g6[...] + b6[...]
        o = jnp.maximum(jnp.dot(xc, lw[...], preferred_element_type=jnp.float32) + lb[...], 0.0)
        ls = o - jnp.max(o, axis=1, keepdims=True)
        ls = ls - jnp.log(jnp.sum(jnp.exp(ls), axis=1, keepdims=True))
        row = jnp.concatenate([ls, jnp.zeros((1, DIM - NCLS), jnp.float32)], axis=1)
        out_r[...] = jnp.concatenate([row, jnp.zeros((7, DIM), jnp.float32)], axis=0)

    full = lambda s: pl.BlockSpec(s, lambda: tuple(0 for _ in s))
    args = [r2, r3, r4,
            p["attn1_W"], p["attn1_b"], p["attn2_W"], p["attn2_b"],
            p["attn3_W"], p["attn3_b"],
            p["bn61_g"], p["bn61_b"], p["bn62_g"], p["bn62_b"],
            p["bn63_g"], p["bn63_b"],
            p["attn_W"], p["attn_b"], p["bn6_g"], p["bn6_b"],
            p["lin1_W"], p["lin1_b"]]
    return pl.pallas_call(
        body,
        in_specs=[full(a.shape) for a in args],
        out_specs=full((8, DIM)),
        out_shape=jax.ShapeDtypeStruct((8, DIM), jnp.float32),
    )(*args)


def kernel(x, edge_index, batch, params):
    p = params
    ones = jnp.ones((DIM,), jnp.float32)
    zeros = jnp.zeros((DIM,), jnp.float32)

    xp = jnp.zeros((N0P, DIM), jnp.float32).at[:N0].set(x)
    pad_e = jnp.full((EP - E,), N0, jnp.int32)
    srcp0 = jnp.concatenate([edge_index[0], pad_e])
    dstp0 = jnp.concatenate([edge_index[1], pad_e])

    def bpad(b):
        return jnp.zeros((8,), jnp.float32).at[0].set(b[0])

    A0, deg0p = _edge_agg(N0P, True)(xp, srcp0, dstp0)
    x0, h0, dinv0, deg0 = _tc_gin(xp, A0, p["conv1_1_W"], p["conv1_1_b"],
                                  p["conv1_2_W"], p["conv1_2_b"], ones, zeros,
                                  N0, extras=(deg0p, p["pool1_W"][:, 0]))
    h0 = h0.reshape(N0P)
    sc0 = _score_scatter(N0P)(srcp0, dstp0, dinv0, h0)
    score0, selp0 = _tc_topk(sc0, dinv0, h0, bpad(p["pool1_b"]), K1, N0)
    nm1, cnm1, perm1, vals1 = _select_compact(N0P, N1P, K1, K1, N0)(score0, selp0)
    srcp1, dstp1 = _edge_remap(N0P, K1)(srcp0, dstp0, nm1)
    xs1 = _row_gather(N1P)(x0, perm1)
    x1 = _tc_pool(xs1, vals1, p["bn1_g"], p["bn1_b"], K1)

    A1, deg1p = _edge_agg(N1P, True)(x1, srcp1, dstp1)
    x1g, h1, dinv1, deg1 = _tc_gin(x1, A1, p["conv2_1_W"], p["conv2_1_b"],
                                   p["conv2_2_W"], p["conv2_2_b"], ones, zeros,
                                   K1, extras=(deg1p, p["pool2_W"][:, 0]))
    h1 = h1.reshape(N1P)
    sc1 = _score_scatter(N1P)(srcp1, dstp1, dinv1, h1)
    score1, selp1 = _tc_topk(sc1, dinv1, h1, bpad(p["pool2_b"]), K2, K1)
    nm2, cnm2, perm2, vals2 = _select_compact(N1P, N2P, K2, K2, K1)(score1, selp1)
    srcp2, dstp2 = _edge_remap(N1P, K2)(srcp1, dstp1, nm2)
    xs2 = _row_gather(N2P)(x1g, perm2)
    x2a = _tc_pool(xs2, vals2, p["bn2_g"], p["bn2_b"], K2)

    A2 = _edge_agg(N2P, False)(x2a, srcp2, dstp2)
    x2 = _tc_gin(x2a, A2, p["conv3_1_W"], p["conv3_1_b"], p["conv3_2_W"],
                 p["conv3_2_b"], p["bn3_g"], p["bn3_b"], K2)[0]

    up1 = _row_gather(N1P)(x2, cnm2)
    S1 = _edge_agg(N1P, False)(up1, srcp1, dstp1)
    x3in = _tc_unpool(up1, S1, nm2, deg1, x1g, p["rw1"], K1)
    A3 = _edge_agg(N1P, False)(x3in, srcp1, dstp1)
    x3 = _tc_gin(x3in, A3, p["conv4_1_W"], p["conv4_1_b"], p["conv4_2_W"],
                 p["conv4_2_b"], p["bn4_g"], p["bn4_b"], K1)[0]

    up0 = _row_gather(N0P)(x3, cnm1)
    S0 = _edge_agg(N0P, False)(up0, srcp0, dstp0)
    x4in = _tc_unpool(up0, S0, nm1, deg0, x0, p["rw2"], N0)
    A4 = _edge_agg(N0P, False)(x4in, srcp0, dstp0)
    x4 = _tc_gin(x4in, A4, p["conv5_1_W"], p["conv5_1_b"], p["conv5_2_W"],
                 p["conv5_2_b"], p["bn5_g"], p["bn5_b"], N0)[0]

    r2 = _tc_read(x2, K2)
    r3 = _tc_read(x3, K1)
    r4 = _tc_read(x4, N0)
    out = _tc_head(r2, r3, r4, p)
    return out[0:1, :NCLS]
